# Initial kernel scaffold; baseline (speedup 1.0000x reference)
import jax
import jax.numpy as jnp
from jax.experimental import pallas as pl


def kernel(feat1, feat2, edge_index, Ws, bs, Wnet, bnet):
    raise NotImplementedError("write your pallas kernel here")


# speedup vs baseline: 220.9630x; 220.9630x over previous
import jax
import jax.numpy as jnp
from jax import lax
from jax.experimental import pallas as pl
from jax.experimental.pallas import tpu as pltpu
from jax.experimental.pallas import tpu_sc as plsc

N = 50000
NP = 50176
E = 3200000
EP = 3211264
NC, NS, LANES = 2, 16, 16
NW = NC * NS
TILE_E = EP // NW
CHUNK = 2048
N_CHUNKS = TILE_E // CHUNK
SL = NP // NS
SLW = NP // NW

f32 = jnp.float32


def _mesh():
    return plsc.VectorSubcoreMesh(core_axis_name="c", subcore_axis_name="s")


def _make_scat(with_count):
    n_out = 3 if with_count else 2
    sds = jax.ShapeDtypeStruct
    out_type = tuple(sds((NC * NP,), f32) for _ in range(n_out))
    scratch = [
        pltpu.VMEM((CHUNK,), jnp.int32),
        pltpu.VMEM((CHUNK,), jnp.int32),
        pltpu.VMEM((CHUNK,), f32),
        pltpu.VMEM((CHUNK,), f32),
        pltpu.VMEM((SL,), f32),
        pltpu.VMEM_SHARED((NP,), f32),
        pltpu.VMEM_SHARED((NP,), f32),
        pltpu.VMEM_SHARED((NP,), f32),
        pltpu.VMEM_SHARED((NP,), f32),
    ]
    if with_count:
        scratch += [
            pltpu.VMEM_SHARED((NP,), f32),
            pltpu.VMEM((CHUNK,), f32),
        ]

    def body(xt0, xt1, src2, dst2, *rest):
        if with_count:
            (outA, outB, outC, srcb, dstb, v0, v1, sliceb,
             tab0, tab1, acc0, acc1, accc, onesb) = rest
            accs = ((acc0, outA), (acc1, outB), (accc, outC))
        else:
            (outA, outB, srcb, dstb, v0, v1, sliceb,
             tab0, tab1, acc0, acc1) = rest
            accs = ((acc0, outA), (acc1, outB))
        c = lax.axis_index("c")
        s = lax.axis_index("s")
        wid = s * NC + c
        ssl = pl.ds(s * SL, SL)

        for xt, tab in ((xt0, tab0), (xt1, tab1)):
            pltpu.sync_copy(xt.at[ssl], sliceb)
            pltpu.sync_copy(sliceb, tab.at[ssl])

        def _z(i, carry):
            sliceb[pl.ds(i * LANES, LANES)] = jnp.zeros((LANES,), f32)
            return carry
        lax.fori_loop(0, SL // LANES, _z, 0)
        for acc, _ in accs:
            pltpu.sync_copy(sliceb, acc.at[ssl])

        if with_count:
            def _o(i, carry):
                onesb[pl.ds(i * LANES, LANES)] = jnp.ones((LANES,), f32)
                return carry
            lax.fori_loop(0, CHUNK // LANES, _o, 0)

        plsc.subcore_barrier()

        e0 = wid * TILE_E

        def _chunk(i, carry):
            r = e0 + i * CHUNK
            pltpu.sync_copy(src2.at[pl.ds(r, CHUNK)], srcb)
            pltpu.sync_copy(dst2.at[pl.ds(r, CHUNK)], dstb)
            pltpu.sync_copy(tab0.at[srcb], v0)
            pltpu.sync_copy(tab1.at[srcb], v1)
            pltpu.sync_copy(v0, acc0.at[dstb], add=True)
            pltpu.sync_copy(v1, acc1.at[dstb], add=True)
            if with_count:
                pltpu.sync_copy(onesb, accc.at[dstb], add=True)
            return carry
        lax.fori_loop(0, N_CHUNKS, _chunk, 0)

        plsc.subcore_barrier()

        for acc, out in accs:
            pltpu.sync_copy(acc.at[ssl], sliceb)
            pltpu.sync_copy(sliceb, out.at[pl.ds(c * NP + s * SL, SL)])

    return pl.kernel(
        body,
        out_type=out_type,
        mesh=_mesh(),
        scratch_types=scratch,
    )


def _make_elem(mode):
    sds = jax.ShapeDtypeStruct
    v = sds((NP,), f32)
    out_type = {1: (v, v, v, v, v), 2: (v, v, v, v), 3: (v, v)}[mode]
    nbuf = 8 if mode == 3 else 6
    scratch = [pltpu.VMEM((SLW,), f32) for _ in range(nbuf)]
    scratch += [pltpu.VMEM((LANES,), f32), pltpu.VMEM((LANES,), f32)]

    def body(*args):
        if mode == 1:
            (x0, x1, aA, aB, aC, wb, bb,
             oo0, oo1, xo0, xo1, inv_out,
             xb, a0b, a1b, invb, ob, xnb, wv, bv) = args
            accs, oouts, xouts = (aA, aB), (oo0, oo1), (xo0, xo1)
        elif mode == 2:
            (x0, x1, aA, aB, inv_in, wb, bb,
             oo0, oo1, xo0, xo1,
             xb, a0b, a1b, invb, ob, xnb, wv, bv) = args
            accs, oouts, xouts = (aA, aB), (oo0, oo1), (xo0, xo1)
        else:
            (x0, x1, aA, aB, inv_in, o1a, o1b, o2a, o2b, wb, bb,
             mo0, mo1,
             xb, a0b, a1b, invb, ob, o1v, o2v, xnb, wv, bv) = args
            accs, oouts = (aA, aB), (mo0, mo1)
        c = lax.axis_index("c")
        s = lax.axis_index("s")
        wid = s * NC + c
        off = wid * SLW
        sl = pl.ds(off, SLW)
        pltpu.sync_copy(wb, wv)
        pltpu.sync_copy(bb, bv)
        w = wv[...]
        b = bv[...]

        if mode == 1:
            pltpu.sync_copy(aC.at[pl.ds(off, SLW)], a0b)
            pltpu.sync_copy(aC.at[pl.ds(NP + off, SLW)], a1b)

            def _iv(i, carry):
                d = pl.ds(i * LANES, LANES)
                cnt = a0b[d] + a1b[d]
                invb[d] = 1.0 / jnp.maximum(cnt, 1.0)
                return carry
            lax.fori_loop(0, SLW // LANES, _iv, 0)
            pltpu.sync_copy(invb, inv_out.at[sl])
        else:
            pltpu.sync_copy(inv_in.at[sl], invb)

        for ch in range(2):
            xin = (x0, x1)[ch]
            acc = accs[ch]
            pltpu.sync_copy(xin.at[sl], xb)
            pltpu.sync_copy(acc.at[pl.ds(off, SLW)], a0b)
            pltpu.sync_copy(acc.at[pl.ds(NP + off, SLW)], a1b)
            if mode == 3:
                pltpu.sync_copy((o1a, o1b)[ch].at[sl], o1v)
                pltpu.sync_copy((o2a, o2b)[ch].at[sl], o2v)

            def _e(i, carry):
                d = pl.ds(i * LANES, LANES)
                ag = (a0b[d] + a1b[d]) * invb[d]
                out = (xb[d] + ag) * w + b
                if mode == 3:
                    ob[d] = (o1v[d] + o2v[d] + out) * (1.0 / 3.0)
                else:
                    ob[d] = out
                    xnb[d] = jnp.maximum(out, 0.0)
                return carry
            lax.fori_loop(0, SLW // LANES, _e, 0)

            pltpu.sync_copy(ob, oouts[ch].at[sl])
            if mode != 3:
                pltpu.sync_copy(xnb, xouts[ch].at[sl])

    return pl.kernel(body, out_type=out_type, mesh=_mesh(),
                     scratch_types=scratch)


_scat_cnt = _make_scat(True)
_scat = _make_scat(False)
_elem1 = _make_elem(1)
_elem2 = _make_elem(2)
_elem3 = _make_elem(3)


_BK = 1024
_KB = NP // _BK


def _tc_body(m_ref, w_ref, b_ref, out_ref, acc_ref):
    k = pl.program_id(0)

    @pl.when(k == 0)
    def _():
        acc_ref[...] = jnp.zeros_like(acc_ref)

    bound = N - k * _BK
    rid = lax.broadcasted_iota(jnp.int32, (_BK, 1024), 0)
    w = jnp.where(rid < bound, w_ref[...], 0.0)
    acc_ref[...] += jnp.dot(m_ref[...], w, preferred_element_type=f32)

    @pl.when(k == _KB - 1)
    def _():
        y0 = acc_ref[0:1, :] + b_ref[...]
        y1 = acc_ref[1:2, :] + b_ref[...]
        out_ref[...] = jax.nn.sigmoid(
            jnp.sum(y0 * y1, axis=1, keepdims=True))


def _tc_matmul(m2, Wnet, bnet2):
    return pl.pallas_call(
        _tc_body,
        grid=(_KB,),
        in_specs=[
            pl.BlockSpec((2, _BK), lambda k: (0, k)),
            pl.BlockSpec((_BK, 1024), lambda k: (k, 0)),
            pl.BlockSpec((1, 1024), lambda k: (0, 0)),
        ],
        out_specs=pl.BlockSpec((1, 1), lambda k: (0, 0)),
        out_shape=jax.ShapeDtypeStruct((1, 1), f32),
        scratch_shapes=[pltpu.VMEM((2, 1024), f32)],
    )(m2, Wnet, bnet2)


def kernel(feat1, feat2, edge_index, Ws, bs, Wnet, bnet):
    src = edge_index[0].astype(jnp.int32)
    dst = edge_index[1].astype(jnp.int32)
    pad = EP - E
    src2 = jnp.concatenate([src, jnp.zeros((pad,), jnp.int32)])
    dst2 = jnp.concatenate([dst, jnp.full((pad,), N, jnp.int32)])
    zpad = jnp.zeros((NP - N,), f32)
    x00 = jnp.concatenate([feat1[:, 0], zpad])
    x01 = jnp.concatenate([feat2[:, 0], zpad])
    wv = [jnp.full((LANES,), Ws[i, 0, 0], f32) for i in range(3)]
    bv = [jnp.full((LANES,), bs[i, 0], f32) for i in range(3)]

    aA, aB, aC = _scat_cnt(x00, x01, src2, dst2)
    o1a, o1b, x2a, x2b, inv = _elem1(x00, x01, aA, aB, aC, wv[0], bv[0])
    aA, aB = _scat(x2a, x2b, src2, dst2)
    o2a, o2b, x3a, x3b = _elem2(x2a, x2b, aA, aB, inv, wv[1], bv[1])
    aA, aB = _scat(x3a, x3b, src2, dst2)
    m0, m1 = _elem3(x3a, x3b, aA, aB, inv, o1a, o1b, o2a, o2b,
                    wv[2], bv[2])

    m2 = jnp.stack([m0, m1])
    return _tc_matmul(m2, Wnet, bnet.reshape(1, 1024))


# speedup vs baseline: 460.2936x; 2.0831x over previous
import jax
import jax.numpy as jnp
from jax import lax
from jax.experimental import pallas as pl
from jax.experimental.pallas import tpu as pltpu
from jax.experimental.pallas import tpu_sc as plsc

N = 50000
NP = 50176
E = 3200000
NC, NS, LANES = 2, 16, 16
NW = NC * NS
CHUNK = 1024
N_CHUNKS = 99
TILE_E = N_CHUNKS * CHUNK
EP = TILE_E * NW
NIB = 6
NVB = 3
SL = NP // NS
SLW = NP // NW

f32 = jnp.float32


def _mesh():
    return plsc.VectorSubcoreMesh(core_axis_name="c", subcore_axis_name="s")


def _make_scat(with_count):
    n_out = 3 if with_count else 2
    sds = jax.ShapeDtypeStruct
    out_type = tuple(sds((NC * NP,), f32) for _ in range(n_out))
    scratch = [
        pltpu.VMEM((NP,), f32),
        pltpu.VMEM((NP,), f32),
    ]
    scratch += [pltpu.VMEM((CHUNK,), jnp.int32) for _ in range(2 * NIB)]
    scratch += [pltpu.VMEM((CHUNK,), f32) for _ in range(2 * NVB)]
    scratch += [pltpu.VMEM((SLW,), f32)]
    scratch += [pltpu.VMEM_SHARED((NP,), f32),
                pltpu.VMEM_SHARED((NP,), f32)]
    if with_count:
        scratch += [
            pltpu.VMEM_SHARED((NP,), f32),
            pltpu.VMEM((CHUNK,), f32),
        ]
    scratch += [pltpu.SemaphoreType.DMA for _ in range(NIB + NVB)]

    def body(xt0, xt1, src2, dst2, *rest):
        outs = rest[:n_out]
        rest = rest[n_out:]
        tab0, tab1 = rest[0], rest[1]
        srcbs = rest[2:2 + NIB]
        dstbs = rest[2 + NIB:2 + 2 * NIB]
        v0s = rest[2 + 2 * NIB:2 + 2 * NIB + NVB]
        v1s = rest[2 + 2 * NIB + NVB:2 + 2 * NIB + 2 * NVB]
        rest = rest[2 + 2 * NIB + 2 * NVB:]
        sliceb = rest[0]
        if with_count:
            acc0, acc1, accc, onesb = rest[1:5]
            sems = rest[5:]
            accs = ((acc0, outs[0]), (acc1, outs[1]), (accc, outs[2]))
        else:
            acc0, acc1 = rest[1:3]
            sems = rest[3:]
            accs = ((acc0, outs[0]), (acc1, outs[1]))
        sem_ix = sems[:NIB]
        sem_sc = sems[NIB:]

        c = lax.axis_index("c")
        s = lax.axis_index("s")
        wid = s * NC + c
        ssl = pl.ds(s * SL, SL)

        pltpu.sync_copy(xt0, tab0)
        pltpu.sync_copy(xt1, tab1)

        def _z(i, carry):
            sliceb[pl.ds(i * LANES, LANES)] = jnp.zeros((LANES,), f32)
            return carry
        lax.fori_loop(0, SLW // LANES, _z, 0)
        for acc, _ in accs:
            for h in range(2):
                pltpu.sync_copy(sliceb, acc.at[pl.ds(s * SL + h * SLW, SLW)])

        if with_count:
            def _o(i, carry):
                onesb[pl.ds(i * LANES, LANES)] = jnp.ones((LANES,), f32)
                return carry
            lax.fori_loop(0, CHUNK // LANES, _o, 0)

        plsc.subcore_barrier()

        e0 = wid * TILE_E

        def issue_idx(cid, k):
            r = e0 + cid * CHUNK
            pltpu.async_copy(src2.at[pl.ds(r, CHUNK)], srcbs[k], sem_ix[k])
            pltpu.async_copy(dst2.at[pl.ds(r, CHUNK)], dstbs[k], sem_ix[k])

        def wait_idx(k):
            pltpu.make_async_copy(src2.at[pl.ds(0, CHUNK)], srcbs[k],
                                  sem_ix[k]).wait()
            pltpu.make_async_copy(src2.at[pl.ds(0, CHUNK)], dstbs[k],
                                  sem_ix[k]).wait()

        def issue_scat(d, k):
            pltpu.async_copy(v0s[d], acc0.at[dstbs[k]], sem_sc[d], add=True)
            pltpu.async_copy(v1s[d], acc1.at[dstbs[k]], sem_sc[d], add=True)
            if with_count:
                pltpu.async_copy(onesb, accc.at[dstbs[k]], sem_sc[d],
                                 add=True)

        def wait_scat(d):
            pltpu.make_async_copy(v0s[d], acc0.at[dstbs[d]], sem_sc[d]).wait()
            pltpu.make_async_copy(v1s[d], acc1.at[dstbs[d]], sem_sc[d]).wait()
            if with_count:
                pltpu.make_async_copy(onesb, accc.at[dstbs[d]],
                                      sem_sc[d]).wait()

        def gather(d, k):
            def _g(i, carry):
                d16 = pl.ds(i * LANES, LANES)
                idx = srcbs[k][d16]
                v0s[d][d16] = plsc.load_gather(tab0, [idx])
                v1s[d][d16] = plsc.load_gather(tab1, [idx])
                return carry
            lax.fori_loop(0, CHUNK // LANES, _g, 0)

        def step(cid, e):
            d = e % NVB
            if isinstance(cid, int):
                if cid >= NVB:
                    wait_scat(d)
                if cid + NVB < N_CHUNKS:
                    issue_idx(cid + NVB, (e + NVB) % NIB)
            else:
                @pl.when(cid >= NVB)
                def _():
                    wait_scat(d)

                @pl.when(cid + NVB < N_CHUNKS)
                def _():
                    issue_idx(cid + NVB, (e + NVB) % NIB)
            wait_idx(e)
            gather(d, e)
            issue_scat(d, e)

        for k in range(NVB):
            issue_idx(k, k)

        def bigbody(j6, carry):
            for e in range(NIB):
                step(j6 * NIB + e, e)
            return carry
        lax.fori_loop(0, N_CHUNKS // NIB, bigbody, 0)
        for e in range(N_CHUNKS % NIB):
            step((N_CHUNKS // NIB) * NIB + e, e)
        for d in range(NVB):
            wait_scat(d)

        plsc.subcore_barrier()

        for acc, out in accs:
            for h in range(2):
                pltpu.sync_copy(acc.at[pl.ds(s * SL + h * SLW, SLW)], sliceb)
                pltpu.sync_copy(
                    sliceb, out.at[pl.ds(c * NP + s * SL + h * SLW, SLW)])

    return pl.kernel(
        body,
        out_type=out_type,
        mesh=_mesh(),
        scratch_types=scratch,
        compiler_params=pltpu.CompilerParams(needs_layout_passes=False),
    )


def _make_elem(mode):
    sds = jax.ShapeDtypeStruct
    v = sds((NP,), f32)
    out_type = {1: (v, v, v, v, v), 2: (v, v, v, v), 3: (v, v)}[mode]
    nbuf = 8 if mode == 3 else 6
    scratch = [pltpu.VMEM((SLW,), f32) for _ in range(nbuf)]
    scratch += [pltpu.VMEM((LANES,), f32), pltpu.VMEM((LANES,), f32)]

    def body(*args):
        if mode == 1:
            (x0, x1, aA, aB, aC, wb, bb,
             oo0, oo1, xo0, xo1, inv_out,
             xb, a0b, a1b, invb, ob, xnb, wv, bv) = args
            accs, oouts, xouts = (aA, aB), (oo0, oo1), (xo0, xo1)
        elif mode == 2:
            (x0, x1, aA, aB, inv_in, wb, bb,
             oo0, oo1, xo0, xo1,
             xb, a0b, a1b, invb, ob, xnb, wv, bv) = args
            accs, oouts, xouts = (aA, aB), (oo0, oo1), (xo0, xo1)
        else:
            (x0, x1, aA, aB, inv_in, o1a, o1b, o2a, o2b, wb, bb,
             mo0, mo1,
             xb, a0b, a1b, invb, ob, o1v, o2v, xnb, wv, bv) = args
            accs, oouts = (aA, aB), (mo0, mo1)
        c = lax.axis_index("c")
        s = lax.axis_index("s")
        wid = s * NC + c
        off = wid * SLW
        sl = pl.ds(off, SLW)
        pltpu.sync_copy(wb, wv)
        pltpu.sync_copy(bb, bv)
        w = wv[...]
        b = bv[...]

        if mode == 1:
            pltpu.sync_copy(aC.at[pl.ds(off, SLW)], a0b)
            pltpu.sync_copy(aC.at[pl.ds(NP + off, SLW)], a1b)

            def _iv(i, carry):
                d = pl.ds(i * LANES, LANES)
                cnt = a0b[d] + a1b[d]
                invb[d] = 1.0 / jnp.maximum(cnt, 1.0)
                return carry
            lax.fori_loop(0, SLW // LANES, _iv, 0)
            pltpu.sync_copy(invb, inv_out.at[sl])
        else:
            pltpu.sync_copy(inv_in.at[sl], invb)

        for ch in range(2):
            xin = (x0, x1)[ch]
            acc = accs[ch]
            pltpu.sync_copy(xin.at[sl], xb)
            pltpu.sync_copy(acc.at[pl.ds(off, SLW)], a0b)
            pltpu.sync_copy(acc.at[pl.ds(NP + off, SLW)], a1b)
            if mode == 3:
                pltpu.sync_copy((o1a, o1b)[ch].at[sl], o1v)
                pltpu.sync_copy((o2a, o2b)[ch].at[sl], o2v)

            def _e(i, carry):
                d = pl.ds(i * LANES, LANES)
                ag = (a0b[d] + a1b[d]) * invb[d]
                out = (xb[d] + ag) * w + b
                if mode == 3:
                    ob[d] = (o1v[d] + o2v[d] + out) * (1.0 / 3.0)
                else:
                    ob[d] = out
                    xnb[d] = jnp.maximum(out, 0.0)
                return carry
            lax.fori_loop(0, SLW // LANES, _e, 0)

            pltpu.sync_copy(ob, oouts[ch].at[sl])
            if mode != 3:
                pltpu.sync_copy(xnb, xouts[ch].at[sl])

    return pl.kernel(body, out_type=out_type, mesh=_mesh(),
                     scratch_types=scratch)


_scat_cnt = _make_scat(True)
_scat = _make_scat(False)
_elem1 = _make_elem(1)
_elem2 = _make_elem(2)
_elem3 = _make_elem(3)


_BK = 1024
_KB = NP // _BK


def _tc_body(m_ref, w_ref, b_ref, out_ref, acc_ref):
    k = pl.program_id(0)

    @pl.when(k == 0)
    def _():
        acc_ref[...] = jnp.zeros_like(acc_ref)

    bound = N - k * _BK
    rid = lax.broadcasted_iota(jnp.int32, (_BK, 1024), 0)
    w = jnp.where(rid < bound, w_ref[...], 0.0)
    acc_ref[...] += jnp.dot(m_ref[...], w, preferred_element_type=f32)

    @pl.when(k == _KB - 1)
    def _():
        y0 = acc_ref[0:1, :] + b_ref[...]
        y1 = acc_ref[1:2, :] + b_ref[...]
        out_ref[...] = jax.nn.sigmoid(
            jnp.sum(y0 * y1, axis=1, keepdims=True))


def _tc_matmul(m2, Wnet, bnet2):
    return pl.pallas_call(
        _tc_body,
        grid=(_KB,),
        in_specs=[
            pl.BlockSpec((2, _BK), lambda k: (0, k)),
            pl.BlockSpec((_BK, 1024), lambda k: (k, 0)),
            pl.BlockSpec((1, 1024), lambda k: (0, 0)),
        ],
        out_specs=pl.BlockSpec((1, 1), lambda k: (0, 0)),
        out_shape=jax.ShapeDtypeStruct((1, 1), f32),
        scratch_shapes=[pltpu.VMEM((2, 1024), f32)],
    )(m2, Wnet, bnet2)


def kernel(feat1, feat2, edge_index, Ws, bs, Wnet, bnet):
    src = edge_index[0].astype(jnp.int32)
    dst = edge_index[1].astype(jnp.int32)
    pad = EP - E
    src2 = jnp.concatenate([src, jnp.zeros((pad,), jnp.int32)])
    trash = N + (jnp.arange(pad, dtype=jnp.int32) % (NP - N))
    dst2 = jnp.concatenate([dst, trash])
    zpad = jnp.zeros((NP - N,), f32)
    x00 = jnp.concatenate([feat1[:, 0], zpad])
    x01 = jnp.concatenate([feat2[:, 0], zpad])
    wv = [jnp.full((LANES,), Ws[i, 0, 0], f32) for i in range(3)]
    bv = [jnp.full((LANES,), bs[i, 0], f32) for i in range(3)]

    aA, aB, aC = _scat_cnt(x00, x01, src2, dst2)
    o1a, o1b, x2a, x2b, inv = _elem1(x00, x01, aA, aB, aC, wv[0], bv[0])
    aA, aB = _scat(x2a, x2b, src2, dst2)
    o2a, o2b, x3a, x3b = _elem2(x2a, x2b, aA, aB, inv, wv[1], bv[1])
    aA, aB = _scat(x3a, x3b, src2, dst2)
    m0, m1 = _elem3(x3a, x3b, aA, aB, inv, o1a, o1b, o2a, o2b,
                    wv[2], bv[2])

    m2 = jnp.stack([m0, m1])
    return _tc_matmul(m2, Wnet, bnet.reshape(1, 1024))


# speedup vs baseline: 486.3438x; 1.0566x over previous
import jax
import jax.numpy as jnp
from jax import lax
from jax.experimental import pallas as pl
from jax.experimental.pallas import tpu as pltpu
from jax.experimental.pallas import tpu_sc as plsc

N = 50000
NP = 50176
E = 3200000
NC, NS, LANES = 2, 16, 16
NW = NC * NS
TILE_E = E // NW
CHUNK = 1280
N_CHUNKS = TILE_E // CHUNK
TAIL = TILE_E - N_CHUNKS * CHUNK
NIB = 4
NVB = 2
SL = NP // NS
SLW = NP // NW

f32 = jnp.float32


def _mesh():
    return plsc.VectorSubcoreMesh(core_axis_name="c", subcore_axis_name="s")


def _make_scat(with_count):
    n_out = 3 if with_count else 2
    sds = jax.ShapeDtypeStruct
    out_type = tuple(sds((NC * NP,), f32) for _ in range(n_out))
    scratch = [
        pltpu.VMEM((N,), f32),
        pltpu.VMEM((N,), f32),
    ]
    scratch += [pltpu.VMEM((CHUNK,), jnp.int32) for _ in range(2 * NIB)]
    scratch += [pltpu.VMEM((CHUNK,), f32) for _ in range(2 * NVB)]
    scratch += [pltpu.VMEM((SLW // 4,), f32)]
    scratch += [pltpu.VMEM_SHARED((NP,), f32),
                pltpu.VMEM_SHARED((NP,), f32)]
    if with_count:
        scratch += [
            pltpu.VMEM_SHARED((NP,), f32),
            pltpu.VMEM((CHUNK,), f32),
        ]
    scratch += [pltpu.VMEM((TAIL,), jnp.int32),
                pltpu.VMEM((TAIL,), jnp.int32),
                pltpu.VMEM((TAIL,), f32),
                pltpu.VMEM((TAIL,), f32)]
    if with_count:
        scratch += [pltpu.VMEM((TAIL,), f32)]
    scratch += [pltpu.SemaphoreType.DMA for _ in range(NIB + NVB + 1)]

    def body(xt0, xt1, src2, dst2, *rest):
        outs = rest[:n_out]
        rest = rest[n_out:]
        tab0, tab1 = rest[0], rest[1]
        srcbs = rest[2:2 + NIB]
        dstbs = rest[2 + NIB:2 + 2 * NIB]
        v0s = rest[2 + 2 * NIB:2 + 2 * NIB + NVB]
        v1s = rest[2 + 2 * NIB + NVB:2 + 2 * NIB + 2 * NVB]
        rest = rest[2 + 2 * NIB + 2 * NVB:]
        sliceb = rest[0]
        if with_count:
            acc0, acc1, accc, onesb = rest[1:5]
            srct, dstt, v0t, v1t, onest = rest[5:10]
            sems = rest[10:]
            accs = ((acc0, outs[0]), (acc1, outs[1]), (accc, outs[2]))
        else:
            acc0, acc1 = rest[1:3]
            srct, dstt, v0t, v1t = rest[3:7]
            sems = rest[7:]
            accs = ((acc0, outs[0]), (acc1, outs[1]))
        sem_ix = sems[:NIB]
        sem_sc = sems[NIB:NIB + NVB]
        sem_t = sems[NIB + NVB]

        c = lax.axis_index("c")
        s = lax.axis_index("s")
        wid = s * NC + c
        ssl = pl.ds(s * SL, SL)

        pltpu.sync_copy(xt0.at[pl.ds(0, N)], tab0)
        pltpu.sync_copy(xt1.at[pl.ds(0, N)], tab1)

        def _z(i, carry):
            sliceb[pl.ds(i * LANES, LANES)] = jnp.zeros((LANES,), f32)
            return carry
        lax.fori_loop(0, SLW // 4 // LANES, _z, 0)
        for acc, _ in accs:
            for h in range(8):
                pltpu.sync_copy(
                    sliceb, acc.at[pl.ds(s * SL + h * (SLW // 4), SLW // 4)])

        if with_count:
            def _o(i, carry):
                onesb[pl.ds(i * LANES, LANES)] = jnp.ones((LANES,), f32)
                return carry
            lax.fori_loop(0, CHUNK // LANES, _o, 0)

            def _ot(i, carry):
                onest[pl.ds(i * LANES, LANES)] = jnp.ones((LANES,), f32)
                return carry
            lax.fori_loop(0, TAIL // LANES, _ot, 0)

        plsc.subcore_barrier()

        e0 = wid * TILE_E

        def issue_idx(cid, k):
            r = e0 + cid * CHUNK
            pltpu.async_copy(src2.at[pl.ds(r, CHUNK)], srcbs[k], sem_ix[k])
            pltpu.async_copy(dst2.at[pl.ds(r, CHUNK)], dstbs[k], sem_ix[k])

        def wait_idx(k):
            pltpu.make_async_copy(src2.at[pl.ds(0, CHUNK)], srcbs[k],
                                  sem_ix[k]).wait()
            pltpu.make_async_copy(src2.at[pl.ds(0, CHUNK)], dstbs[k],
                                  sem_ix[k]).wait()

        def issue_scat(d, k):
            pltpu.async_copy(v0s[d], acc0.at[dstbs[k]], sem_sc[d], add=True)
            pltpu.async_copy(v1s[d], acc1.at[dstbs[k]], sem_sc[d], add=True)
            if with_count:
                pltpu.async_copy(onesb, accc.at[dstbs[k]], sem_sc[d],
                                 add=True)

        def wait_scat(d):
            pltpu.make_async_copy(v0s[d], acc0.at[dstbs[d]], sem_sc[d]).wait()
            pltpu.make_async_copy(v1s[d], acc1.at[dstbs[d]], sem_sc[d]).wait()
            if with_count:
                pltpu.make_async_copy(onesb, accc.at[dstbs[d]],
                                      sem_sc[d]).wait()

        def gather(d, k):
            def _g(i, carry):
                d16 = pl.ds(i * LANES, LANES)
                idx = srcbs[k][d16]
                v0s[d][d16] = plsc.load_gather(tab0, [idx])
                v1s[d][d16] = plsc.load_gather(tab1, [idx])
                return carry
            lax.fori_loop(0, CHUNK // LANES, _g, 0)

        def step(cid, e):
            d = e % NVB
            if isinstance(cid, int):
                if cid >= NVB:
                    wait_scat(d)
                if cid + NVB < N_CHUNKS:
                    issue_idx(cid + NVB, (e + NVB) % NIB)
            else:
                @pl.when(cid >= NVB)
                def _():
                    wait_scat(d)

                @pl.when(cid + NVB < N_CHUNKS)
                def _():
                    issue_idx(cid + NVB, (e + NVB) % NIB)
            wait_idx(e)
            gather(d, e)
            issue_scat(d, e)

        for k in range(NVB):
            issue_idx(k, k)
        rt = e0 + N_CHUNKS * CHUNK
        pltpu.async_copy(src2.at[pl.ds(rt, TAIL)], srct, sem_t)
        pltpu.async_copy(dst2.at[pl.ds(rt, TAIL)], dstt, sem_t)

        def bigbody(j6, carry):
            for e in range(NIB):
                step(j6 * NIB + e, e)
            return carry
        lax.fori_loop(0, N_CHUNKS // NIB, bigbody, 0)
        for e in range(N_CHUNKS % NIB):
            step((N_CHUNKS // NIB) * NIB + e, e)

        pltpu.make_async_copy(src2.at[pl.ds(0, TAIL)], srct, sem_t).wait()
        pltpu.make_async_copy(src2.at[pl.ds(0, TAIL)], dstt, sem_t).wait()

        def _gt(i, carry):
            d16 = pl.ds(i * LANES, LANES)
            idx = srct[d16]
            v0t[d16] = plsc.load_gather(tab0, [idx])
            v1t[d16] = plsc.load_gather(tab1, [idx])
            return carry
        lax.fori_loop(0, TAIL // LANES, _gt, 0)
        pltpu.async_copy(v0t, acc0.at[dstt], sem_t, add=True)
        pltpu.async_copy(v1t, acc1.at[dstt], sem_t, add=True)
        if with_count:
            pltpu.async_copy(onest, accc.at[dstt], sem_t, add=True)

        for d in range(NVB):
            wait_scat(d)
        pltpu.make_async_copy(v0t, acc0.at[dstt], sem_t).wait()
        pltpu.make_async_copy(v1t, acc1.at[dstt], sem_t).wait()
        if with_count:
            pltpu.make_async_copy(onest, accc.at[dstt], sem_t).wait()

        plsc.subcore_barrier()

        for acc, out in accs:
            for h in range(8):
                hh = h * (SLW // 4)
                pltpu.sync_copy(acc.at[pl.ds(s * SL + hh, SLW // 4)], sliceb)
                pltpu.sync_copy(
                    sliceb, out.at[pl.ds(c * NP + s * SL + hh, SLW // 4)])

    return pl.kernel(
        body,
        out_type=out_type,
        mesh=_mesh(),
        scratch_types=scratch,
        compiler_params=pltpu.CompilerParams(needs_layout_passes=False),
    )


def _make_elem(mode):
    sds = jax.ShapeDtypeStruct
    v = sds((NP,), f32)
    out_type = {1: (v, v, v, v, v), 2: (v, v, v, v), 3: (v, v)}[mode]
    nbuf = 8 if mode == 3 else 6
    scratch = [pltpu.VMEM((SLW,), f32) for _ in range(nbuf)]
    scratch += [pltpu.VMEM((LANES,), f32), pltpu.VMEM((LANES,), f32)]

    def body(*args):
        if mode == 1:
            (x0, x1, aA, aB, aC, wb, bb,
             oo0, oo1, xo0, xo1, inv_out,
             xb, a0b, a1b, invb, ob, xnb, wv, bv) = args
            accs, oouts, xouts = (aA, aB), (oo0, oo1), (xo0, xo1)
        elif mode == 2:
            (x0, x1, aA, aB, inv_in, wb, bb,
             oo0, oo1, xo0, xo1,
             xb, a0b, a1b, invb, ob, xnb, wv, bv) = args
            accs, oouts, xouts = (aA, aB), (oo0, oo1), (xo0, xo1)
        else:
            (x0, x1, aA, aB, inv_in, o1a, o1b, o2a, o2b, wb, bb,
             mo0, mo1,
             xb, a0b, a1b, invb, ob, o1v, o2v, xnb, wv, bv) = args
            accs, oouts = (aA, aB), (mo0, mo1)
        c = lax.axis_index("c")
        s = lax.axis_index("s")
        wid = s * NC + c
        off = wid * SLW
        sl = pl.ds(off, SLW)
        pltpu.sync_copy(wb, wv)
        pltpu.sync_copy(bb, bv)
        w = wv[...]
        b = bv[...]

        if mode == 1:
            pltpu.sync_copy(aC.at[pl.ds(off, SLW)], a0b)
            pltpu.sync_copy(aC.at[pl.ds(NP + off, SLW)], a1b)

            def _iv(i, carry):
                d = pl.ds(i * LANES, LANES)
                cnt = a0b[d] + a1b[d]
                invb[d] = 1.0 / jnp.maximum(cnt, 1.0)
                return carry
            lax.fori_loop(0, SLW // LANES, _iv, 0)
            pltpu.sync_copy(invb, inv_out.at[sl])
        else:
            pltpu.sync_copy(inv_in.at[sl], invb)

        for ch in range(2):
            xin = (x0, x1)[ch]
            acc = accs[ch]
            pltpu.sync_copy(xin.at[sl], xb)
            pltpu.sync_copy(acc.at[pl.ds(off, SLW)], a0b)
            pltpu.sync_copy(acc.at[pl.ds(NP + off, SLW)], a1b)
            if mode == 3:
                pltpu.sync_copy((o1a, o1b)[ch].at[sl], o1v)
                pltpu.sync_copy((o2a, o2b)[ch].at[sl], o2v)

            def _e(i, carry):
                d = pl.ds(i * LANES, LANES)
                ag = (a0b[d] + a1b[d]) * invb[d]
                out = (xb[d] + ag) * w + b
                if mode == 3:
                    ob[d] = (o1v[d] + o2v[d] + out) * (1.0 / 3.0)
                else:
                    ob[d] = out
                    xnb[d] = jnp.maximum(out, 0.0)
                return carry
            lax.fori_loop(0, SLW // LANES, _e, 0)

            pltpu.sync_copy(ob, oouts[ch].at[sl])
            if mode != 3:
                pltpu.sync_copy(xnb, xouts[ch].at[sl])

    return pl.kernel(body, out_type=out_type, mesh=_mesh(),
                     scratch_types=scratch)


_scat_cnt = _make_scat(True)
_scat = _make_scat(False)
_elem1 = _make_elem(1)
_elem2 = _make_elem(2)


_BK = 1024
_KB = NP // _BK


def _tc_body(x3a, x3b, aA, aB, inv, o1a, o1b, o2a, o2b, w3, b3,
             w_ref, bn_ref, out_ref, acc_ref):
    k = pl.program_id(0)

    @pl.when(k == 0)
    def _():
        acc_ref[...] = jnp.zeros_like(acc_ref)

    row = pl.ds(k, 1)
    row2 = pl.ds(_KB + k, 1)
    invr = inv[row, :]
    w3v = w3[...]
    b3v = b3[...]
    ms = []
    for x3, o1, o2 in ((x3a, o1a, o2a), (x3b, o1b, o2b)):
        a = aA[row, :] + aA[row2, :] if x3 is x3a else aB[row, :] + aB[row2, :]
        out3 = (x3[row, :] + a * invr) * w3v + b3v
        ms.append((o1[row, :] + o2[row, :] + out3) * (1.0 / 3.0))
    m2 = jnp.concatenate(ms, axis=0)

    bound = N - k * _BK
    rid = lax.broadcasted_iota(jnp.int32, (_BK, 1024), 0)
    w = jnp.where(rid < bound, w_ref[...], 0.0)
    acc_ref[...] += jnp.dot(m2, w, preferred_element_type=f32)

    @pl.when(k == _KB - 1)
    def _():
        y0 = acc_ref[0:1, :] + bn_ref[...]
        y1 = acc_ref[1:2, :] + bn_ref[...]
        out_ref[...] = jax.nn.sigmoid(
            jnp.sum(y0 * y1, axis=1, keepdims=True))


def _tc_final(x3a, x3b, aA, aB, inv, o1a, o1b, o2a, o2b, w3, b3,
              Wnet, bnet2):
    small = pl.BlockSpec((_KB, _BK), lambda k: (0, 0))
    part = pl.BlockSpec((2 * _KB, _BK), lambda k: (0, 0))
    vec = pl.BlockSpec((1, 1024), lambda k: (0, 0))
    return pl.pallas_call(
        _tc_body,
        grid=(_KB,),
        in_specs=[small, small, part, part, small, small, small, small,
                  small, vec, vec,
                  pl.BlockSpec((_BK, 1024), lambda k: (k, 0)), vec],
        out_specs=pl.BlockSpec((1, 1), lambda k: (0, 0)),
        out_shape=jax.ShapeDtypeStruct((1, 1), f32),
        scratch_shapes=[pltpu.VMEM((2, 1024), f32)],
    )(x3a, x3b, aA, aB, inv, o1a, o1b, o2a, o2b, w3, b3, Wnet, bnet2)


def kernel(feat1, feat2, edge_index, Ws, bs, Wnet, bnet):
    src = edge_index[0].astype(jnp.int32)
    dst = edge_index[1].astype(jnp.int32)
    zpad = jnp.zeros((NP - N,), f32)
    x00 = jnp.concatenate([feat1[:, 0], zpad])
    x01 = jnp.concatenate([feat2[:, 0], zpad])
    wv = [jnp.full((LANES,), Ws[i, 0, 0], f32) for i in range(2)]
    bv = [jnp.full((LANES,), bs[i, 0], f32) for i in range(2)]

    aA, aB, aC = _scat_cnt(x00, x01, src, dst)
    o1a, o1b, x2a, x2b, inv = _elem1(x00, x01, aA, aB, aC, wv[0], bv[0])
    aA, aB = _scat(x2a, x2b, src, dst)
    o2a, o2b, x3a, x3b = _elem2(x2a, x2b, aA, aB, inv, wv[1], bv[1])
    aA, aB = _scat(x3a, x3b, src, dst)

    def rs(v):
        return v.reshape(_KB, _BK)

    def rs2(v):
        return v.reshape(2 * _KB, _BK)

    return _tc_final(rs(x3a), rs(x3b), rs2(aA), rs2(aB), rs(inv),
                     rs(o1a), rs(o1b), rs(o2a), rs(o2b),
                     jnp.full((1, 1024), Ws[2, 0, 0], f32),
                     jnp.full((1, 1024), bs[2, 0], f32),
                     Wnet, bnet.reshape(1, 1024))


# speedup vs baseline: 497.6255x; 1.0232x over previous
import jax
import jax.numpy as jnp
from jax import lax
from jax.experimental import pallas as pl
from jax.experimental.pallas import tpu as pltpu
from jax.experimental.pallas import tpu_sc as plsc

N = 50000
NP = 50176
E = 3200000
NC, NS, LANES = 2, 16, 16
NW = NC * NS
TILE_E = E // NW
CHUNK = 1280
N_CHUNKS = TILE_E // CHUNK
TAIL = TILE_E - N_CHUNKS * CHUNK
NIB = 4
NVB = 2
SL = NP // NS
SLW = NP // NW

f32 = jnp.float32


def _mesh():
    return plsc.VectorSubcoreMesh(core_axis_name="c", subcore_axis_name="s")


def _make_scat(with_count):
    n_out = 3 if with_count else 2
    sds = jax.ShapeDtypeStruct
    out_type = tuple(sds((NC * NP,), f32) for _ in range(n_out))
    scratch = [
        pltpu.VMEM((N,), f32),
        pltpu.VMEM((N,), f32),
    ]
    scratch += [pltpu.VMEM((CHUNK,), jnp.int32) for _ in range(2 * NIB)]
    scratch += [pltpu.VMEM((CHUNK,), f32) for _ in range(2 * NVB)]
    scratch += [pltpu.VMEM((SLW,), f32)]
    scratch += [pltpu.VMEM_SHARED((NP,), f32),
                pltpu.VMEM_SHARED((NP,), f32)]
    if with_count:
        scratch += [
            pltpu.VMEM_SHARED((NP,), f32),
            pltpu.VMEM((CHUNK,), f32),
        ]
    scratch += [pltpu.VMEM((TAIL,), jnp.int32),
                pltpu.VMEM((TAIL,), jnp.int32),
                pltpu.VMEM((TAIL,), f32),
                pltpu.VMEM((TAIL,), f32)]
    if with_count:
        scratch += [pltpu.VMEM((TAIL,), f32)]
    scratch += [pltpu.SemaphoreType.DMA for _ in range(NIB + NVB + 1)]

    def body(xt0, xt1, src2, dst2, *rest):
        outs = rest[:n_out]
        rest = rest[n_out:]
        tab0, tab1 = rest[0], rest[1]
        srcbs = rest[2:2 + NIB]
        dstbs = rest[2 + NIB:2 + 2 * NIB]
        v0s = rest[2 + 2 * NIB:2 + 2 * NIB + NVB]
        v1s = rest[2 + 2 * NIB + NVB:2 + 2 * NIB + 2 * NVB]
        rest = rest[2 + 2 * NIB + 2 * NVB:]
        sliceb = rest[0]
        if with_count:
            acc0, acc1, accc, onesb = rest[1:5]
            srct, dstt, v0t, v1t, onest = rest[5:10]
            sems = rest[10:]
            accs = ((acc0, outs[0]), (acc1, outs[1]), (accc, outs[2]))
        else:
            acc0, acc1 = rest[1:3]
            srct, dstt, v0t, v1t = rest[3:7]
            sems = rest[7:]
            accs = ((acc0, outs[0]), (acc1, outs[1]))
        sem_ix = sems[:NIB]
        sem_sc = sems[NIB:NIB + NVB]
        sem_t = sems[NIB + NVB]

        c = lax.axis_index("c")
        s = lax.axis_index("s")
        wid = s * NC + c
        ssl = pl.ds(s * SL, SL)

        pltpu.sync_copy(xt0.at[pl.ds(0, N)], tab0)
        pltpu.sync_copy(xt1.at[pl.ds(0, N)], tab1)

        def _z(i, carry):
            sliceb[pl.ds(i * LANES, LANES)] = jnp.zeros((LANES,), f32)
            return carry
        lax.fori_loop(0, SLW // LANES, _z, 0)
        for acc, _ in accs:
            for h in range(2):
                pltpu.async_copy(
                    sliceb, acc.at[pl.ds(s * SL + h * SLW, SLW)], sem_t)
        for acc, _ in accs:
            for h in range(2):
                pltpu.make_async_copy(
                    sliceb, acc.at[pl.ds(s * SL + h * SLW, SLW)],
                    sem_t).wait()

        if with_count:
            def _o(i, carry):
                onesb[pl.ds(i * LANES, LANES)] = jnp.ones((LANES,), f32)
                return carry
            lax.fori_loop(0, CHUNK // LANES, _o, 0)

            def _ot(i, carry):
                onest[pl.ds(i * LANES, LANES)] = jnp.ones((LANES,), f32)
                return carry
            lax.fori_loop(0, TAIL // LANES, _ot, 0)

        plsc.subcore_barrier()

        e0 = wid * TILE_E

        def issue_idx(cid, k):
            r = e0 + cid * CHUNK
            pltpu.async_copy(src2.at[pl.ds(r, CHUNK)], srcbs[k], sem_ix[k])
            pltpu.async_copy(dst2.at[pl.ds(r, CHUNK)], dstbs[k], sem_ix[k])

        def wait_idx(k):
            pltpu.make_async_copy(src2.at[pl.ds(0, CHUNK)], srcbs[k],
                                  sem_ix[k]).wait()
            pltpu.make_async_copy(src2.at[pl.ds(0, CHUNK)], dstbs[k],
                                  sem_ix[k]).wait()

        def issue_scat(d, k):
            pltpu.async_copy(v0s[d], acc0.at[dstbs[k]], sem_sc[d], add=True)
            pltpu.async_copy(v1s[d], acc1.at[dstbs[k]], sem_sc[d], add=True)
            if with_count:
                pltpu.async_copy(onesb, accc.at[dstbs[k]], sem_sc[d],
                                 add=True)

        def wait_scat(d):
            pltpu.make_async_copy(v0s[d], acc0.at[dstbs[d]], sem_sc[d]).wait()
            pltpu.make_async_copy(v1s[d], acc1.at[dstbs[d]], sem_sc[d]).wait()
            if with_count:
                pltpu.make_async_copy(onesb, accc.at[dstbs[d]],
                                      sem_sc[d]).wait()

        def gather(d, k):
            def _g(i, carry):
                d16 = pl.ds(i * LANES, LANES)
                idx = srcbs[k][d16]
                v0s[d][d16] = plsc.load_gather(tab0, [idx])
                v1s[d][d16] = plsc.load_gather(tab1, [idx])
                return carry
            lax.fori_loop(0, CHUNK // LANES, _g, 0)

        def step(cid, e):
            d = e % NVB
            if isinstance(cid, int):
                if cid >= NVB:
                    wait_scat(d)
                if cid + NVB < N_CHUNKS:
                    issue_idx(cid + NVB, (e + NVB) % NIB)
            else:
                @pl.when(cid >= NVB)
                def _():
                    wait_scat(d)

                @pl.when(cid + NVB < N_CHUNKS)
                def _():
                    issue_idx(cid + NVB, (e + NVB) % NIB)
            wait_idx(e)
            gather(d, e)
            issue_scat(d, e)

        for k in range(NVB):
            issue_idx(k, k)
        rt = e0 + N_CHUNKS * CHUNK
        pltpu.async_copy(src2.at[pl.ds(rt, TAIL)], srct, sem_t)
        pltpu.async_copy(dst2.at[pl.ds(rt, TAIL)], dstt, sem_t)

        def bigbody(j6, carry):
            for e in range(NIB):
                step(j6 * NIB + e, e)
            return carry
        lax.fori_loop(0, N_CHUNKS // NIB, bigbody, 0)
        for e in range(N_CHUNKS % NIB):
            step((N_CHUNKS // NIB) * NIB + e, e)

        pltpu.make_async_copy(src2.at[pl.ds(0, TAIL)], srct, sem_t).wait()
        pltpu.make_async_copy(src2.at[pl.ds(0, TAIL)], dstt, sem_t).wait()

        def _gt(i, carry):
            d16 = pl.ds(i * LANES, LANES)
            idx = srct[d16]
            v0t[d16] = plsc.load_gather(tab0, [idx])
            v1t[d16] = plsc.load_gather(tab1, [idx])
            return carry
        lax.fori_loop(0, TAIL // LANES, _gt, 0)
        pltpu.async_copy(v0t, acc0.at[dstt], sem_t, add=True)
        pltpu.async_copy(v1t, acc1.at[dstt], sem_t, add=True)
        if with_count:
            pltpu.async_copy(onest, accc.at[dstt], sem_t, add=True)

        for d in range(NVB):
            wait_scat(d)
        pltpu.make_async_copy(v0t, acc0.at[dstt], sem_t).wait()
        pltpu.make_async_copy(v1t, acc1.at[dstt], sem_t).wait()
        if with_count:
            pltpu.make_async_copy(onest, accc.at[dstt], sem_t).wait()

        plsc.subcore_barrier()

        for acc, out in accs:
            for h in range(2):
                hh = h * SLW
                pltpu.sync_copy(acc.at[pl.ds(s * SL + hh, SLW)], sliceb)
                pltpu.sync_copy(
                    sliceb, out.at[pl.ds(c * NP + s * SL + hh, SLW)])

    return pl.kernel(
        body,
        out_type=out_type,
        mesh=_mesh(),
        scratch_types=scratch,
        compiler_params=pltpu.CompilerParams(needs_layout_passes=False),
    )


def _make_elem(mode):
    sds = jax.ShapeDtypeStruct
    v = sds((NP,), f32)
    out_type = {1: (v, v, v, v, v), 2: (v, v, v, v), 3: (v, v)}[mode]
    nbuf = 8 if mode == 3 else 6
    scratch = [pltpu.VMEM((SLW,), f32) for _ in range(nbuf)]
    scratch += [pltpu.VMEM((LANES,), f32), pltpu.VMEM((LANES,), f32)]

    def body(*args):
        if mode == 1:
            (x0, x1, aA, aB, aC, wb, bb,
             oo0, oo1, xo0, xo1, inv_out,
             xb, a0b, a1b, invb, ob, xnb, wv, bv) = args
            accs, oouts, xouts = (aA, aB), (oo0, oo1), (xo0, xo1)
        elif mode == 2:
            (x0, x1, aA, aB, inv_in, wb, bb,
             oo0, oo1, xo0, xo1,
             xb, a0b, a1b, invb, ob, xnb, wv, bv) = args
            accs, oouts, xouts = (aA, aB), (oo0, oo1), (xo0, xo1)
        else:
            (x0, x1, aA, aB, inv_in, o1a, o1b, o2a, o2b, wb, bb,
             mo0, mo1,
             xb, a0b, a1b, invb, ob, o1v, o2v, xnb, wv, bv) = args
            accs, oouts = (aA, aB), (mo0, mo1)
        c = lax.axis_index("c")
        s = lax.axis_index("s")
        wid = s * NC + c
        off = wid * SLW
        sl = pl.ds(off, SLW)
        pltpu.sync_copy(wb, wv)
        pltpu.sync_copy(bb, bv)
        w = wv[...]
        b = bv[...]

        if mode == 1:
            pltpu.sync_copy(aC.at[pl.ds(off, SLW)], a0b)
            pltpu.sync_copy(aC.at[pl.ds(NP + off, SLW)], a1b)

            def _iv(i, carry):
                d = pl.ds(i * LANES, LANES)
                cnt = a0b[d] + a1b[d]
                invb[d] = 1.0 / jnp.maximum(cnt, 1.0)
                return carry
            lax.fori_loop(0, SLW // LANES, _iv, 0)
            pltpu.sync_copy(invb, inv_out.at[sl])
        else:
            pltpu.sync_copy(inv_in.at[sl], invb)

        for ch in range(2):
            xin = (x0, x1)[ch]
            acc = accs[ch]
            pltpu.sync_copy(xin.at[sl], xb)
            pltpu.sync_copy(acc.at[pl.ds(off, SLW)], a0b)
            pltpu.sync_copy(acc.at[pl.ds(NP + off, SLW)], a1b)
            if mode == 3:
                pltpu.sync_copy((o1a, o1b)[ch].at[sl], o1v)
                pltpu.sync_copy((o2a, o2b)[ch].at[sl], o2v)

            def _e(i, carry):
                d = pl.ds(i * LANES, LANES)
                ag = (a0b[d] + a1b[d]) * invb[d]
                out = (xb[d] + ag) * w + b
                if mode == 3:
                    ob[d] = (o1v[d] + o2v[d] + out) * (1.0 / 3.0)
                else:
                    ob[d] = out
                    xnb[d] = jnp.maximum(out, 0.0)
                return carry
            lax.fori_loop(0, SLW // LANES, _e, 0)

            pltpu.sync_copy(ob, oouts[ch].at[sl])
            if mode != 3:
                pltpu.sync_copy(xnb, xouts[ch].at[sl])

    return pl.kernel(body, out_type=out_type, mesh=_mesh(),
                     scratch_types=scratch)


_scat_cnt = _make_scat(True)
_scat = _make_scat(False)
_elem1 = _make_elem(1)
_elem2 = _make_elem(2)


_BK = 1024
_KB = NP // _BK


def _tc_body(x3a, x3b, aA, aB, inv, o1a, o1b, o2a, o2b, w3, b3,
             w_ref, bn_ref, out_ref, acc_ref):
    k = pl.program_id(0)

    @pl.when(k == 0)
    def _():
        acc_ref[...] = jnp.zeros_like(acc_ref)

    row = pl.ds(k, 1)
    row2 = pl.ds(_KB + k, 1)
    invr = inv[row, :]
    w3v = w3[...]
    b3v = b3[...]
    ms = []
    for x3, o1, o2 in ((x3a, o1a, o2a), (x3b, o1b, o2b)):
        a = aA[row, :] + aA[row2, :] if x3 is x3a else aB[row, :] + aB[row2, :]
        out3 = (x3[row, :] + a * invr) * w3v + b3v
        ms.append((o1[row, :] + o2[row, :] + out3) * (1.0 / 3.0))
    m2 = jnp.concatenate(ms, axis=0)

    bound = N - k * _BK
    rid = lax.broadcasted_iota(jnp.int32, (_BK, 1024), 0)
    w = jnp.where(rid < bound, w_ref[...], 0.0)
    acc_ref[...] += jnp.dot(m2, w, preferred_element_type=f32)

    @pl.when(k == _KB - 1)
    def _():
        y0 = acc_ref[0:1, :] + bn_ref[...]
        y1 = acc_ref[1:2, :] + bn_ref[...]
        out_ref[...] = jax.nn.sigmoid(
            jnp.sum(y0 * y1, axis=1, keepdims=True))


def _tc_final(x3a, x3b, aA, aB, inv, o1a, o1b, o2a, o2b, w3, b3,
              Wnet, bnet2):
    small = pl.BlockSpec((_KB, _BK), lambda k: (0, 0))
    part = pl.BlockSpec((2 * _KB, _BK), lambda k: (0, 0))
    vec = pl.BlockSpec((1, 1024), lambda k: (0, 0))
    return pl.pallas_call(
        _tc_body,
        grid=(_KB,),
        in_specs=[small, small, part, part, small, small, small, small,
                  small, vec, vec,
                  pl.BlockSpec((_BK, 1024), lambda k: (k, 0)), vec],
        out_specs=pl.BlockSpec((1, 1), lambda k: (0, 0)),
        out_shape=jax.ShapeDtypeStruct((1, 1), f32),
        scratch_shapes=[pltpu.VMEM((2, 1024), f32)],
    )(x3a, x3b, aA, aB, inv, o1a, o1b, o2a, o2b, w3, b3, Wnet, bnet2)


def kernel(feat1, feat2, edge_index, Ws, bs, Wnet, bnet):
    src = edge_index[0].astype(jnp.int32)
    dst = edge_index[1].astype(jnp.int32)
    zpad = jnp.zeros((NP - N,), f32)
    x00 = jnp.concatenate([feat1[:, 0], zpad])
    x01 = jnp.concatenate([feat2[:, 0], zpad])
    wv = [jnp.full((LANES,), Ws[i, 0, 0], f32) for i in range(2)]
    bv = [jnp.full((LANES,), bs[i, 0], f32) for i in range(2)]

    aA, aB, aC = _scat_cnt(x00, x01, src, dst)
    o1a, o1b, x2a, x2b, inv = _elem1(x00, x01, aA, aB, aC, wv[0], bv[0])
    aA, aB = _scat(x2a, x2b, src, dst)
    o2a, o2b, x3a, x3b = _elem2(x2a, x2b, aA, aB, inv, wv[1], bv[1])
    aA, aB = _scat(x3a, x3b, src, dst)

    def rs(v):
        return v.reshape(_KB, _BK)

    def rs2(v):
        return v.reshape(2 * _KB, _BK)

    return _tc_final(rs(x3a), rs(x3b), rs2(aA), rs2(aB), rs(inv),
                     rs(o1a), rs(o1b), rs(o2a), rs(o2b),
                     jnp.full((1, 1024), Ws[2, 0, 0], f32),
                     jnp.full((1, 1024), bs[2, 0], f32),
                     Wnet, bnet.reshape(1, 1024))


# speedup vs baseline: 505.8509x; 1.0165x over previous
import jax
import jax.numpy as jnp
from jax import lax
from jax.experimental import pallas as pl
from jax.experimental.pallas import tpu as pltpu
from jax.experimental.pallas import tpu_sc as plsc

N = 50000
NP = 50176
E = 3200000
NC, NS, LANES = 2, 16, 16
NW = NC * NS
TILE_E = E // NW
CHUNK = 1280
N_CHUNKS = TILE_E // CHUNK
TAIL = TILE_E - N_CHUNKS * CHUNK
NIB = 4
NVB = 2
SL = NP // NS
SLW = NP // NW

f32 = jnp.float32


def _mesh():
    return plsc.VectorSubcoreMesh(core_axis_name="c", subcore_axis_name="s")


def _make_scat(with_count):
    n_out = 3 if with_count else 2
    sds = jax.ShapeDtypeStruct
    out_type = tuple(sds((NC * NP,), f32) for _ in range(n_out))
    scratch = [
        pltpu.VMEM((N,), f32),
        pltpu.VMEM((N,), f32),
    ]
    scratch += [pltpu.VMEM((CHUNK,), jnp.int32) for _ in range(2 * NIB)]
    scratch += [pltpu.VMEM((CHUNK,), f32) for _ in range(2 * NVB)]
    scratch += [pltpu.VMEM((SLW,), f32)]
    scratch += [pltpu.VMEM_SHARED((NP,), f32),
                pltpu.VMEM_SHARED((NP,), f32)]
    if with_count:
        scratch += [
            pltpu.VMEM_SHARED((NP,), f32),
            pltpu.VMEM((CHUNK,), f32),
        ]
    scratch += [pltpu.VMEM((TAIL,), jnp.int32),
                pltpu.VMEM((TAIL,), jnp.int32),
                pltpu.VMEM((TAIL,), f32),
                pltpu.VMEM((TAIL,), f32)]
    if with_count:
        scratch += [pltpu.VMEM((TAIL,), f32)]
    scratch += [pltpu.SemaphoreType.DMA for _ in range(NIB + NVB + 1)]

    def body(xt0, xt1, src2, dst2, *rest):
        outs = rest[:n_out]
        rest = rest[n_out:]
        tab0, tab1 = rest[0], rest[1]
        srcbs = rest[2:2 + NIB]
        dstbs = rest[2 + NIB:2 + 2 * NIB]
        v0s = rest[2 + 2 * NIB:2 + 2 * NIB + NVB]
        v1s = rest[2 + 2 * NIB + NVB:2 + 2 * NIB + 2 * NVB]
        rest = rest[2 + 2 * NIB + 2 * NVB:]
        sliceb = rest[0]
        if with_count:
            acc0, acc1, accc, onesb = rest[1:5]
            srct, dstt, v0t, v1t, onest = rest[5:10]
            sems = rest[10:]
            accs = ((acc0, outs[0]), (acc1, outs[1]), (accc, outs[2]))
        else:
            acc0, acc1 = rest[1:3]
            srct, dstt, v0t, v1t = rest[3:7]
            sems = rest[7:]
            accs = ((acc0, outs[0]), (acc1, outs[1]))
        sem_ix = sems[:NIB]
        sem_sc = sems[NIB:NIB + NVB]
        sem_t = sems[NIB + NVB]

        c = lax.axis_index("c")
        s = lax.axis_index("s")
        wid = s * NC + c
        ssl = pl.ds(s * SL, SL)

        pltpu.sync_copy(xt0.at[pl.ds(0, N)], tab0)
        pltpu.sync_copy(xt1.at[pl.ds(0, N)], tab1)

        def _z(i, carry):
            sliceb[pl.ds(i * LANES, LANES)] = jnp.zeros((LANES,), f32)
            return carry
        lax.fori_loop(0, SLW // LANES, _z, 0)
        for acc, _ in accs:
            for h in range(2):
                pltpu.async_copy(
                    sliceb, acc.at[pl.ds(s * SL + h * SLW, SLW)], sem_t)
        for acc, _ in accs:
            for h in range(2):
                pltpu.make_async_copy(
                    sliceb, acc.at[pl.ds(s * SL + h * SLW, SLW)],
                    sem_t).wait()

        if with_count:
            def _o(i, carry):
                onesb[pl.ds(i * LANES, LANES)] = jnp.ones((LANES,), f32)
                return carry
            lax.fori_loop(0, CHUNK // LANES, _o, 0)

            def _ot(i, carry):
                onest[pl.ds(i * LANES, LANES)] = jnp.ones((LANES,), f32)
                return carry
            lax.fori_loop(0, TAIL // LANES, _ot, 0)

        plsc.subcore_barrier()

        e0 = wid * TILE_E

        def issue_idx(cid, k):
            r = e0 + cid * CHUNK
            pltpu.async_copy(src2.at[pl.ds(r, CHUNK)], srcbs[k], sem_ix[k])
            pltpu.async_copy(dst2.at[pl.ds(r, CHUNK)], dstbs[k], sem_ix[k])

        def wait_idx(k):
            pltpu.make_async_copy(src2.at[pl.ds(0, CHUNK)], srcbs[k],
                                  sem_ix[k]).wait()
            pltpu.make_async_copy(src2.at[pl.ds(0, CHUNK)], dstbs[k],
                                  sem_ix[k]).wait()

        def issue_scat(d, k):
            pltpu.async_copy(v0s[d], acc0.at[dstbs[k]], sem_sc[d], add=True)
            pltpu.async_copy(v1s[d], acc1.at[dstbs[k]], sem_sc[d], add=True)
            if with_count:
                pltpu.async_copy(onesb, accc.at[dstbs[k]], sem_sc[d],
                                 add=True)

        def wait_scat(d):
            pltpu.make_async_copy(v0s[d], acc0.at[dstbs[d]], sem_sc[d]).wait()
            pltpu.make_async_copy(v1s[d], acc1.at[dstbs[d]], sem_sc[d]).wait()
            if with_count:
                pltpu.make_async_copy(onesb, accc.at[dstbs[d]],
                                      sem_sc[d]).wait()

        def gather(d, k):
            def _g(i, carry):
                d16 = pl.ds(i * LANES, LANES)
                idx = srcbs[k][d16]
                v0s[d][d16] = plsc.load_gather(tab0, [idx])
                v1s[d][d16] = plsc.load_gather(tab1, [idx])
                return carry
            lax.fori_loop(0, CHUNK // LANES, _g, 0)

        def step(cid, e):
            d = e % NVB
            if isinstance(cid, int):
                if cid >= NVB:
                    wait_scat(d)
                if cid + NVB < N_CHUNKS:
                    issue_idx(cid + NVB, (e + NVB) % NIB)
            else:
                @pl.when(cid >= NVB)
                def _():
                    wait_scat(d)

                @pl.when(cid + NVB < N_CHUNKS)
                def _():
                    issue_idx(cid + NVB, (e + NVB) % NIB)
            wait_idx(e)
            gather(d, e)
            issue_scat(d, e)

        for k in range(NVB):
            issue_idx(k, k)
        rt = e0 + N_CHUNKS * CHUNK
        pltpu.async_copy(src2.at[pl.ds(rt, TAIL)], srct, sem_t)
        pltpu.async_copy(dst2.at[pl.ds(rt, TAIL)], dstt, sem_t)

        def bigbody(j6, carry):
            for e in range(NIB):
                step(j6 * NIB + e, e)
            return carry
        lax.fori_loop(0, N_CHUNKS // NIB, bigbody, 0)
        for e in range(N_CHUNKS % NIB):
            step((N_CHUNKS // NIB) * NIB + e, e)

        pltpu.make_async_copy(src2.at[pl.ds(0, TAIL)], srct, sem_t).wait()
        pltpu.make_async_copy(src2.at[pl.ds(0, TAIL)], dstt, sem_t).wait()

        def _gt(i, carry):
            d16 = pl.ds(i * LANES, LANES)
            idx = srct[d16]
            v0t[d16] = plsc.load_gather(tab0, [idx])
            v1t[d16] = plsc.load_gather(tab1, [idx])
            return carry
        lax.fori_loop(0, TAIL // LANES, _gt, 0)
        pltpu.async_copy(v0t, acc0.at[dstt], sem_t, add=True)
        pltpu.async_copy(v1t, acc1.at[dstt], sem_t, add=True)
        if with_count:
            pltpu.async_copy(onest, accc.at[dstt], sem_t, add=True)

        for d in range(NVB):
            wait_scat(d)
        pltpu.make_async_copy(v0t, acc0.at[dstt], sem_t).wait()
        pltpu.make_async_copy(v1t, acc1.at[dstt], sem_t).wait()
        if with_count:
            pltpu.make_async_copy(onest, accc.at[dstt], sem_t).wait()

        plsc.subcore_barrier()

        for acc, out in accs:
            for h in range(2):
                hh = h * SLW
                pltpu.sync_copy(acc.at[pl.ds(s * SL + hh, SLW)], sliceb)
                pltpu.sync_copy(
                    sliceb, out.at[pl.ds(c * NP + s * SL + hh, SLW)])

    return pl.kernel(
        body,
        out_type=out_type,
        mesh=_mesh(),
        scratch_types=scratch,
        compiler_params=pltpu.CompilerParams(needs_layout_passes=False),
    )


def _make_elem(mode):
    sds = jax.ShapeDtypeStruct
    v = sds((NP,), f32)
    out_type = {1: (v, v, v, v, v), 2: (v, v, v, v)}[mode]
    scratch = [pltpu.VMEM((SLW,), f32) for _ in range(13)]
    scratch += [pltpu.VMEM((LANES,), f32), pltpu.VMEM((LANES,), f32),
                pltpu.SemaphoreType.DMA]

    def body(*args):
        if mode == 1:
            (x0, x1, aA, aB, aC, wb, bb,
             oo0, oo1, xo0, xo1, inv_out, *rest) = args
        else:
            (x0, x1, aA, aB, inv_in, wb, bb,
             oo0, oo1, xo0, xo1, *rest) = args
        (xb0, xb1, p00, p01, p10, p11, cb0, cb1, invb,
         ob0, ob1, xnb0, xnb1, wv, bv, sem) = rest
        c = lax.axis_index("c")
        s = lax.axis_index("s")
        wid = s * NC + c
        off = wid * SLW
        sl = pl.ds(off, SLW)

        loads = [(wb, wv), (bb, bv),
                 (x0.at[sl], xb0), (x1.at[sl], xb1),
                 (aA.at[pl.ds(off, SLW)], p00),
                 (aA.at[pl.ds(NP + off, SLW)], p01),
                 (aB.at[pl.ds(off, SLW)], p10),
                 (aB.at[pl.ds(NP + off, SLW)], p11)]
        if mode == 1:
            loads += [(aC.at[pl.ds(off, SLW)], cb0),
                      (aC.at[pl.ds(NP + off, SLW)], cb1)]
        else:
            loads += [(inv_in.at[sl], invb)]
        for wave in (loads[:5], loads[5:]):
            for src, dstb in wave:
                pltpu.async_copy(src, dstb, sem)
            for src, dstb in wave:
                pltpu.make_async_copy(src, dstb, sem).wait()

        w = wv[...]
        b = bv[...]

        if mode == 1:
            def _iv(i, carry):
                d = pl.ds(i * LANES, LANES)
                invb[d] = 1.0 / jnp.maximum(cb0[d] + cb1[d], 1.0)
                return carry
            lax.fori_loop(0, SLW // LANES, _iv, 0)

        def _e(i, carry):
            d = pl.ds(i * LANES, LANES)
            iv = invb[d]
            out_a = (xb0[d] + (p00[d] + p01[d]) * iv) * w + b
            out_b = (xb1[d] + (p10[d] + p11[d]) * iv) * w + b
            ob0[d] = out_a
            ob1[d] = out_b
            xnb0[d] = jnp.maximum(out_a, 0.0)
            xnb1[d] = jnp.maximum(out_b, 0.0)
            return carry
        lax.fori_loop(0, SLW // LANES, _e, 0)

        stores = [(ob0, oo0.at[sl]), (ob1, oo1.at[sl]),
                  (xnb0, xo0.at[sl]), (xnb1, xo1.at[sl])]
        if mode == 1:
            stores += [(invb, inv_out.at[sl])]
        for srcb, dstr in stores:
            pltpu.async_copy(srcb, dstr, sem)
        for srcb, dstr in stores:
            pltpu.make_async_copy(srcb, dstr, sem).wait()

    return pl.kernel(body, out_type=out_type, mesh=_mesh(),
                     scratch_types=scratch,
                     compiler_params=pltpu.CompilerParams(
                         needs_layout_passes=False))


_scat_cnt = _make_scat(True)
_scat = _make_scat(False)
_elem1 = _make_elem(1)
_elem2 = _make_elem(2)


_BK = 1024
_KB = NP // _BK


def _tc_body(x3a, x3b, aA, aB, inv, o1a, o1b, o2a, o2b, w3, b3,
             w_ref, bn_ref, out_ref, acc_ref):
    k = pl.program_id(0)

    @pl.when(k == 0)
    def _():
        acc_ref[...] = jnp.zeros_like(acc_ref)

    row = pl.ds(k, 1)
    row2 = pl.ds(_KB + k, 1)
    invr = inv[row, :]
    w3v = w3[...]
    b3v = b3[...]
    ms = []
    for x3, o1, o2 in ((x3a, o1a, o2a), (x3b, o1b, o2b)):
        a = aA[row, :] + aA[row2, :] if x3 is x3a else aB[row, :] + aB[row2, :]
        out3 = (x3[row, :] + a * invr) * w3v + b3v
        ms.append((o1[row, :] + o2[row, :] + out3) * (1.0 / 3.0))
    m2 = jnp.concatenate(ms, axis=0)

    bound = N - k * _BK
    rid = lax.broadcasted_iota(jnp.int32, (_BK, 1024), 0)
    w = jnp.where(rid < bound, w_ref[...], 0.0)
    acc_ref[...] += jnp.dot(m2, w, preferred_element_type=f32)

    @pl.when(k == _KB - 1)
    def _():
        y0 = acc_ref[0:1, :] + bn_ref[...]
        y1 = acc_ref[1:2, :] + bn_ref[...]
        out_ref[...] = jax.nn.sigmoid(
            jnp.sum(y0 * y1, axis=1, keepdims=True))


def _tc_final(x3a, x3b, aA, aB, inv, o1a, o1b, o2a, o2b, w3, b3,
              Wnet, bnet2):
    small = pl.BlockSpec((_KB, _BK), lambda k: (0, 0))
    part = pl.BlockSpec((2 * _KB, _BK), lambda k: (0, 0))
    vec = pl.BlockSpec((1, 1024), lambda k: (0, 0))
    return pl.pallas_call(
        _tc_body,
        grid=(_KB,),
        in_specs=[small, small, part, part, small, small, small, small,
                  small, vec, vec,
                  pl.BlockSpec((_BK, 1024), lambda k: (k, 0)), vec],
        out_specs=pl.BlockSpec((1, 1), lambda k: (0, 0)),
        out_shape=jax.ShapeDtypeStruct((1, 1), f32),
        scratch_shapes=[pltpu.VMEM((2, 1024), f32)],
    )(x3a, x3b, aA, aB, inv, o1a, o1b, o2a, o2b, w3, b3, Wnet, bnet2)


def kernel(feat1, feat2, edge_index, Ws, bs, Wnet, bnet):
    src = edge_index[0].astype(jnp.int32)
    dst = edge_index[1].astype(jnp.int32)
    zpad = jnp.zeros((NP - N,), f32)
    x00 = jnp.concatenate([feat1[:, 0], zpad])
    x01 = jnp.concatenate([feat2[:, 0], zpad])
    wv = [jnp.full((LANES,), Ws[i, 0, 0], f32) for i in range(2)]
    bv = [jnp.full((LANES,), bs[i, 0], f32) for i in range(2)]

    aA, aB, aC = _scat_cnt(x00, x01, src, dst)
    o1a, o1b, x2a, x2b, inv = _elem1(x00, x01, aA, aB, aC, wv[0], bv[0])
    aA, aB = _scat(x2a, x2b, src, dst)
    o2a, o2b, x3a, x3b = _elem2(x2a, x2b, aA, aB, inv, wv[1], bv[1])
    aA, aB = _scat(x3a, x3b, src, dst)

    def rs(v):
        return v.reshape(_KB, _BK)

    def rs2(v):
        return v.reshape(2 * _KB, _BK)

    return _tc_final(rs(x3a), rs(x3b), rs2(aA), rs2(aB), rs(inv),
                     rs(o1a), rs(o1b), rs(o2a), rs(o2b),
                     jnp.full((1, 1024), Ws[2, 0, 0], f32),
                     jnp.full((1, 1024), bs[2, 0], f32),
                     Wnet, bnet.reshape(1, 1024))


# speedup vs baseline: 507.2266x; 1.0027x over previous
import jax
import jax.numpy as jnp
from jax import lax
from jax.experimental import pallas as pl
from jax.experimental.pallas import tpu as pltpu
from jax.experimental.pallas import tpu_sc as plsc

N = 50000
NP = 50176
E = 3200000
NC, NS, LANES = 2, 16, 16
NW = NC * NS
TILE_E = E // NW
CHUNK = 1280
N_CHUNKS = TILE_E // CHUNK
TAIL = TILE_E - N_CHUNKS * CHUNK
NIB = 4
NVB = 2
SL = NP // NS
SLW = NP // NW

f32 = jnp.float32


def _mesh():
    return plsc.VectorSubcoreMesh(core_axis_name="c", subcore_axis_name="s")


def _make_scat(with_count):
    n_out = 3 if with_count else 2
    sds = jax.ShapeDtypeStruct
    out_type = tuple(sds((NC * NP,), f32) for _ in range(n_out))
    scratch = [
        pltpu.VMEM((N,), f32),
        pltpu.VMEM((N,), f32),
    ]
    scratch += [pltpu.VMEM((CHUNK,), jnp.int32) for _ in range(2 * NIB)]
    scratch += [pltpu.VMEM((CHUNK,), f32) for _ in range(2 * NVB)]
    scratch += [pltpu.VMEM((SLW,), f32),
                pltpu.VMEM((SLW,), f32)]
    scratch += [pltpu.VMEM_SHARED((NP,), f32),
                pltpu.VMEM_SHARED((NP,), f32)]
    if with_count:
        scratch += [
            pltpu.VMEM_SHARED((NP,), f32),
            pltpu.VMEM((CHUNK,), f32),
        ]
    scratch += [pltpu.VMEM((TAIL,), jnp.int32),
                pltpu.VMEM((TAIL,), jnp.int32),
                pltpu.VMEM((TAIL,), f32),
                pltpu.VMEM((TAIL,), f32)]
    if with_count:
        scratch += [pltpu.VMEM((TAIL,), f32)]
    scratch += [pltpu.SemaphoreType.DMA for _ in range(NIB + NVB + 1)]

    def body(xt0, xt1, src2, dst2, *rest):
        outs = rest[:n_out]
        rest = rest[n_out:]
        tab0, tab1 = rest[0], rest[1]
        srcbs = rest[2:2 + NIB]
        dstbs = rest[2 + NIB:2 + 2 * NIB]
        v0s = rest[2 + 2 * NIB:2 + 2 * NIB + NVB]
        v1s = rest[2 + 2 * NIB + NVB:2 + 2 * NIB + 2 * NVB]
        rest = rest[2 + 2 * NIB + 2 * NVB:]
        sliceb, sliceb2 = rest[0], rest[1]
        if with_count:
            acc0, acc1, accc, onesb = rest[2:6]
            srct, dstt, v0t, v1t, onest = rest[6:11]
            sems = rest[11:]
            accs = ((acc0, outs[0]), (acc1, outs[1]), (accc, outs[2]))
        else:
            acc0, acc1 = rest[2:4]
            srct, dstt, v0t, v1t = rest[4:8]
            sems = rest[8:]
            accs = ((acc0, outs[0]), (acc1, outs[1]))
        sem_ix = sems[:NIB]
        sem_sc = sems[NIB:NIB + NVB]
        sem_t = sems[NIB + NVB]

        c = lax.axis_index("c")
        s = lax.axis_index("s")
        wid = s * NC + c
        ssl = pl.ds(s * SL, SL)

        pltpu.sync_copy(xt0.at[pl.ds(0, N)], tab0)
        pltpu.sync_copy(xt1.at[pl.ds(0, N)], tab1)

        def _z(i, carry):
            sliceb[pl.ds(i * LANES, LANES)] = jnp.zeros((LANES,), f32)
            return carry
        lax.fori_loop(0, SLW // LANES, _z, 0)
        for acc, _ in accs:
            for h in range(2):
                pltpu.async_copy(
                    sliceb, acc.at[pl.ds(s * SL + h * SLW, SLW)], sem_t)
        for acc, _ in accs:
            for h in range(2):
                pltpu.make_async_copy(
                    sliceb, acc.at[pl.ds(s * SL + h * SLW, SLW)],
                    sem_t).wait()

        if with_count:
            def _o(i, carry):
                onesb[pl.ds(i * LANES, LANES)] = jnp.ones((LANES,), f32)
                return carry
            lax.fori_loop(0, CHUNK // LANES, _o, 0)

            def _ot(i, carry):
                onest[pl.ds(i * LANES, LANES)] = jnp.ones((LANES,), f32)
                return carry
            lax.fori_loop(0, TAIL // LANES, _ot, 0)

        plsc.subcore_barrier()

        e0 = wid * TILE_E

        def issue_idx(cid, k):
            r = e0 + cid * CHUNK
            pltpu.async_copy(src2.at[pl.ds(r, CHUNK)], srcbs[k], sem_ix[k])
            pltpu.async_copy(dst2.at[pl.ds(r, CHUNK)], dstbs[k], sem_ix[k])

        def wait_idx(k):
            pltpu.make_async_copy(src2.at[pl.ds(0, CHUNK)], srcbs[k],
                                  sem_ix[k]).wait()
            pltpu.make_async_copy(src2.at[pl.ds(0, CHUNK)], dstbs[k],
                                  sem_ix[k]).wait()

        def issue_scat(d, k):
            pltpu.async_copy(v0s[d], acc0.at[dstbs[k]], sem_sc[d], add=True)
            pltpu.async_copy(v1s[d], acc1.at[dstbs[k]], sem_sc[d], add=True)
            if with_count:
                pltpu.async_copy(onesb, accc.at[dstbs[k]], sem_sc[d],
                                 add=True)

        def wait_scat(d):
            pltpu.make_async_copy(v0s[d], acc0.at[dstbs[d]], sem_sc[d]).wait()
            pltpu.make_async_copy(v1s[d], acc1.at[dstbs[d]], sem_sc[d]).wait()
            if with_count:
                pltpu.make_async_copy(onesb, accc.at[dstbs[d]],
                                      sem_sc[d]).wait()

        def gather(d, k):
            def _g(i, carry):
                d16 = pl.ds(i * LANES, LANES)
                idx = srcbs[k][d16]
                v0s[d][d16] = plsc.load_gather(tab0, [idx])
                v1s[d][d16] = plsc.load_gather(tab1, [idx])
                return carry
            lax.fori_loop(0, CHUNK // LANES, _g, 0)

        def step(cid, e):
            d = e % NVB
            if isinstance(cid, int):
                if cid >= NVB:
                    wait_scat(d)
                if cid + NVB < N_CHUNKS:
                    issue_idx(cid + NVB, (e + NVB) % NIB)
            else:
                @pl.when(cid >= NVB)
                def _():
                    wait_scat(d)

                @pl.when(cid + NVB < N_CHUNKS)
                def _():
                    issue_idx(cid + NVB, (e + NVB) % NIB)
            wait_idx(e)
            gather(d, e)
            issue_scat(d, e)

        for k in range(NVB):
            issue_idx(k, k)
        rt = e0 + N_CHUNKS * CHUNK
        pltpu.async_copy(src2.at[pl.ds(rt, TAIL)], srct, sem_t)
        pltpu.async_copy(dst2.at[pl.ds(rt, TAIL)], dstt, sem_t)

        def bigbody(j6, carry):
            for e in range(NIB):
                step(j6 * NIB + e, e)
            return carry
        lax.fori_loop(0, N_CHUNKS // NIB, bigbody, 0)
        for e in range(N_CHUNKS % NIB):
            step((N_CHUNKS // NIB) * NIB + e, e)

        pltpu.make_async_copy(src2.at[pl.ds(0, TAIL)], srct, sem_t).wait()
        pltpu.make_async_copy(src2.at[pl.ds(0, TAIL)], dstt, sem_t).wait()

        def _gt(i, carry):
            d16 = pl.ds(i * LANES, LANES)
            idx = srct[d16]
            v0t[d16] = plsc.load_gather(tab0, [idx])
            v1t[d16] = plsc.load_gather(tab1, [idx])
            return carry
        lax.fori_loop(0, TAIL // LANES, _gt, 0)
        pltpu.async_copy(v0t, acc0.at[dstt], sem_t, add=True)
        pltpu.async_copy(v1t, acc1.at[dstt], sem_t, add=True)
        if with_count:
            pltpu.async_copy(onest, accc.at[dstt], sem_t, add=True)

        for d in range(NVB):
            wait_scat(d)
        pltpu.make_async_copy(v0t, acc0.at[dstt], sem_t).wait()
        pltpu.make_async_copy(v1t, acc1.at[dstt], sem_t).wait()
        if with_count:
            pltpu.make_async_copy(onest, accc.at[dstt], sem_t).wait()

        plsc.subcore_barrier()

        pairs = []
        for acc, out in accs:
            for h in range(2):
                hh = h * SLW
                pairs.append((acc.at[pl.ds(s * SL + hh, SLW)],
                              out.at[pl.ds(c * NP + s * SL + hh, SLW)]))
        bufs = (sliceb, sliceb2)
        for i, (src_r, dst_r) in enumerate(pairs):
            b = bufs[i % 2]
            if i >= 2:
                pltpu.make_async_copy(b, pairs[i - 2][1], sem_t).wait()
            pltpu.sync_copy(src_r, b)
            pltpu.async_copy(b, dst_r, sem_t)
        for i in range(len(pairs) - 2, len(pairs)):
            pltpu.make_async_copy(bufs[i % 2], pairs[i][1], sem_t).wait()

    return pl.kernel(
        body,
        out_type=out_type,
        mesh=_mesh(),
        scratch_types=scratch,
        compiler_params=pltpu.CompilerParams(needs_layout_passes=False),
    )


def _make_elem(mode):
    sds = jax.ShapeDtypeStruct
    v = sds((NP,), f32)
    out_type = {1: (v, v, v, v, v), 2: (v, v, v, v)}[mode]
    scratch = [pltpu.VMEM((SLW,), f32) for _ in range(13)]
    scratch += [pltpu.VMEM((LANES,), f32), pltpu.VMEM((LANES,), f32),
                pltpu.SemaphoreType.DMA]

    def body(*args):
        if mode == 1:
            (x0, x1, aA, aB, aC, wb, bb,
             oo0, oo1, xo0, xo1, inv_out, *rest) = args
        else:
            (x0, x1, aA, aB, inv_in, wb, bb,
             oo0, oo1, xo0, xo1, *rest) = args
        (xb0, xb1, p00, p01, p10, p11, cb0, cb1, invb,
         ob0, ob1, xnb0, xnb1, wv, bv, sem) = rest
        c = lax.axis_index("c")
        s = lax.axis_index("s")
        wid = s * NC + c
        off = wid * SLW
        sl = pl.ds(off, SLW)

        loads = [(wb, wv), (bb, bv),
                 (x0.at[sl], xb0), (x1.at[sl], xb1),
                 (aA.at[pl.ds(off, SLW)], p00),
                 (aA.at[pl.ds(NP + off, SLW)], p01),
                 (aB.at[pl.ds(off, SLW)], p10),
                 (aB.at[pl.ds(NP + off, SLW)], p11)]
        if mode == 1:
            loads += [(aC.at[pl.ds(off, SLW)], cb0),
                      (aC.at[pl.ds(NP + off, SLW)], cb1)]
        else:
            loads += [(inv_in.at[sl], invb)]
        for wave in (loads[:5], loads[5:]):
            for src, dstb in wave:
                pltpu.async_copy(src, dstb, sem)
            for src, dstb in wave:
                pltpu.make_async_copy(src, dstb, sem).wait()

        w = wv[...]
        b = bv[...]

        if mode == 1:
            def _iv(i, carry):
                d = pl.ds(i * LANES, LANES)
                invb[d] = 1.0 / jnp.maximum(cb0[d] + cb1[d], 1.0)
                return carry
            lax.fori_loop(0, SLW // LANES, _iv, 0)

        def _e(i, carry):
            d = pl.ds(i * LANES, LANES)
            iv = invb[d]
            out_a = (xb0[d] + (p00[d] + p01[d]) * iv) * w + b
            out_b = (xb1[d] + (p10[d] + p11[d]) * iv) * w + b
            ob0[d] = out_a
            ob1[d] = out_b
            xnb0[d] = jnp.maximum(out_a, 0.0)
            xnb1[d] = jnp.maximum(out_b, 0.0)
            return carry
        lax.fori_loop(0, SLW // LANES, _e, 0)

        stores = [(ob0, oo0.at[sl]), (ob1, oo1.at[sl]),
                  (xnb0, xo0.at[sl]), (xnb1, xo1.at[sl])]
        if mode == 1:
            stores += [(invb, inv_out.at[sl])]
        for srcb, dstr in stores:
            pltpu.async_copy(srcb, dstr, sem)
        for srcb, dstr in stores:
            pltpu.make_async_copy(srcb, dstr, sem).wait()

    return pl.kernel(body, out_type=out_type, mesh=_mesh(),
                     scratch_types=scratch,
                     compiler_params=pltpu.CompilerParams(
                         needs_layout_passes=False))


_scat_cnt = _make_scat(True)
_scat = _make_scat(False)
_elem1 = _make_elem(1)
_elem2 = _make_elem(2)


_BK = 1024
_KB = NP // _BK


def _tc_body(x3a, x3b, aA, aB, inv, o1a, o1b, o2a, o2b, w3, b3,
             w_ref, bn_ref, out_ref, acc_ref):
    k = pl.program_id(0)

    @pl.when(k == 0)
    def _():
        acc_ref[...] = jnp.zeros_like(acc_ref)

    row = pl.ds(k, 1)
    row2 = pl.ds(_KB + k, 1)
    invr = inv[row, :]
    w3v = w3[...]
    b3v = b3[...]
    ms = []
    for x3, o1, o2 in ((x3a, o1a, o2a), (x3b, o1b, o2b)):
        a = aA[row, :] + aA[row2, :] if x3 is x3a else aB[row, :] + aB[row2, :]
        out3 = (x3[row, :] + a * invr) * w3v + b3v
        ms.append((o1[row, :] + o2[row, :] + out3) * (1.0 / 3.0))
    m2 = jnp.concatenate(ms, axis=0)

    bound = N - k * _BK
    rid = lax.broadcasted_iota(jnp.int32, (_BK, 1024), 0)
    w = jnp.where(rid < bound, w_ref[...], 0.0)
    acc_ref[...] += jnp.dot(m2, w, preferred_element_type=f32)

    @pl.when(k == _KB - 1)
    def _():
        y0 = acc_ref[0:1, :] + bn_ref[...]
        y1 = acc_ref[1:2, :] + bn_ref[...]
        out_ref[...] = jax.nn.sigmoid(
            jnp.sum(y0 * y1, axis=1, keepdims=True))


def _tc_final(x3a, x3b, aA, aB, inv, o1a, o1b, o2a, o2b, w3, b3,
              Wnet, bnet2):
    small = pl.BlockSpec((_KB, _BK), lambda k: (0, 0))
    part = pl.BlockSpec((2 * _KB, _BK), lambda k: (0, 0))
    vec = pl.BlockSpec((1, 1024), lambda k: (0, 0))
    return pl.pallas_call(
        _tc_body,
        grid=(_KB,),
        in_specs=[small, small, part, part, small, small, small, small,
                  small, vec, vec,
                  pl.BlockSpec((_BK, 1024), lambda k: (k, 0)), vec],
        out_specs=pl.BlockSpec((1, 1), lambda k: (0, 0)),
        out_shape=jax.ShapeDtypeStruct((1, 1), f32),
        scratch_shapes=[pltpu.VMEM((2, 1024), f32)],
    )(x3a, x3b, aA, aB, inv, o1a, o1b, o2a, o2b, w3, b3, Wnet, bnet2)


def kernel(feat1, feat2, edge_index, Ws, bs, Wnet, bnet):
    src = edge_index[0].astype(jnp.int32)
    dst = edge_index[1].astype(jnp.int32)
    zpad = jnp.zeros((NP - N,), f32)
    x00 = jnp.concatenate([feat1[:, 0], zpad])
    x01 = jnp.concatenate([feat2[:, 0], zpad])
    wv = [jnp.full((LANES,), Ws[i, 0, 0], f32) for i in range(2)]
    bv = [jnp.full((LANES,), bs[i, 0], f32) for i in range(2)]

    aA, aB, aC = _scat_cnt(x00, x01, src, dst)
    o1a, o1b, x2a, x2b, inv = _elem1(x00, x01, aA, aB, aC, wv[0], bv[0])
    aA, aB = _scat(x2a, x2b, src, dst)
    o2a, o2b, x3a, x3b = _elem2(x2a, x2b, aA, aB, inv, wv[1], bv[1])
    aA, aB = _scat(x3a, x3b, src, dst)

    def rs(v):
        return v.reshape(_KB, _BK)

    def rs2(v):
        return v.reshape(2 * _KB, _BK)

    return _tc_final(rs(x3a), rs(x3b), rs2(aA), rs2(aB), rs(inv),
                     rs(o1a), rs(o1b), rs(o2a), rs(o2b),
                     jnp.full((1, 1024), Ws[2, 0, 0], f32),
                     jnp.full((1, 1024), bs[2, 0], f32),
                     Wnet, bnet.reshape(1, 1024))
